# baseline (speedup 1.0000x reference)
import functools

import jax
import jax.numpy as jnp
from jax import lax
from jax.experimental import pallas as pl
from jax.experimental.pallas import tpu as pltpu
from jax.experimental.pallas import tpu_sc as plsc

_CHUNK = 65536


def _make_sc_copy(n):
    info = plsc.get_sparse_core_info()
    nc, ns = info.num_cores, info.num_subcores
    nw = nc * ns
    seg = n // nw
    steps = seg // _CHUNK
    mesh = plsc.VectorSubcoreMesh(core_axis_name="c", subcore_axis_name="s")

    @functools.partial(
        pl.kernel,
        mesh=mesh,
        out_type=[
            jax.ShapeDtypeStruct((3, n), jnp.float32),
            jax.ShapeDtypeStruct((3, n), jnp.float32),
            jax.ShapeDtypeStruct((n,), jnp.float32),
        ],
        scratch_types=[pltpu.VMEM((1, _CHUNK), jnp.float32),
                       pltpu.VMEM((_CHUNK,), jnp.float32)],
    )
    def k(x_hbm, r_hbm, d_hbm, xo_hbm, ro_hbm, do_hbm, buf2, buf1):
        wid = lax.axis_index("s") * nc + lax.axis_index("c")
        base = wid * seg
        for src, dst in ((x_hbm, xo_hbm), (r_hbm, ro_hbm)):
            for row in range(3):
                for c in range(steps):
                    sl = (pl.ds(row, 1), pl.ds(base + c * _CHUNK, _CHUNK))
                    pltpu.sync_copy(src.at[sl], buf2)
                    pltpu.sync_copy(buf2, dst.at[sl])
        for c in range(steps):
            sl = pl.ds(base + c * _CHUNK, _CHUNK)
            pltpu.sync_copy(d_hbm.at[sl], buf1)
            pltpu.sync_copy(buf1, do_hbm.at[sl])

    return k


def kernel(sampled_point_xyz, sampled_point_ray_direction, sampled_point_distance):
    n = sampled_point_xyz.shape[0]
    xt = sampled_point_xyz.T
    rt = sampled_point_ray_direction.T
    pos_t, ray_t, dists = _make_sc_copy(n)(xt, rt, sampled_point_distance)
    return (pos_t.T, ray_t.T, dists)
